# baseline (speedup 1.0000x reference)
import functools

import jax
import jax.numpy as jnp
from jax import lax
from jax.experimental import pallas as pl
from jax.experimental.pallas import tpu as pltpu
from jax.experimental.pallas import tpu_sc as plsc

N_NODE = 4096
K = 16
HIDDEN = 128
B = 512
BIG = 0x7F000000


def _topk_mlp_body(pos_ref, posT_ref, w0a_ref, c0_ref, w1t_ref, b1_ref,
                   w2_ref, b2_ref, idx_ref, s_ref):
    i = pl.program_id(0)
    pb = pos_ref[...]
    pt = posT_ref[...]
    g = jnp.dot(pb, pt, preferred_element_type=jnp.float32,
                precision=lax.Precision.HIGHEST)
    ni = jnp.sum(pb * pb, axis=1, keepdims=True)
    nj = jnp.sum(pt * pt, axis=0, keepdims=True)
    d = jnp.maximum((ni + nj) - (g + g), 0.0)
    db = lax.bitcast_convert_type(d, jnp.int32)
    col = lax.broadcasted_iota(jnp.int32, (B, N_NODE), 1)
    row = lax.broadcasted_iota(jnp.int32, (B, N_NODE), 0) + i * B
    dp = (db & jnp.int32(-4096)) | col
    dp = jnp.where(col == row, jnp.int32(BIG), dp)
    dpf = lax.bitcast_convert_type(dp, jnp.float32)
    bigf = lax.bitcast_convert_type(jnp.int32(BIG), jnp.float32)
    m = None
    ms = []
    ss = []
    for p in range(K):
        if p == 0:
            m = jnp.min(dpf, axis=1, keepdims=True)
        else:
            m = jnp.min(jnp.where(dpf > m, dpf, bigf), axis=1, keepdims=True)
        ms.append(m)
        mi = lax.bitcast_convert_type(m, jnp.int32)
        rad = lax.bitcast_convert_type(mi & jnp.int32(-4096), jnp.float32)
        h = rad * w0a_ref[...] + c0_ref[...]
        h = h * jax.nn.sigmoid(h)
        h = h * lax.rsqrt(jnp.mean(h * h, axis=1, keepdims=True) + 1e-6)
        h = jnp.dot(h.astype(jnp.bfloat16), w1t_ref[...],
                    preferred_element_type=jnp.float32) + b1_ref[...]
        h = h * jax.nn.sigmoid(h)
        h = h * lax.rsqrt(jnp.mean(h * h, axis=1, keepdims=True) + 1e-6)
        ss.append(jnp.sum(h * w2_ref[...], axis=1, keepdims=True) + b2_ref[...])
    v = lax.bitcast_convert_type(jnp.concatenate(ms, axis=1), jnp.int32)
    idx_ref[...] = v & jnp.int32(4095)
    s_ref[...] = jnp.concatenate(ss, axis=1)


def _sc_apply_build():
    mesh = plsc.VectorSubcoreMesh(core_axis_name="c", subcore_axis_name="s")
    n_workers = 32
    r_per_w = N_NODE // n_workers

    @functools.partial(
        pl.kernel, mesh=mesh,
        compiler_params=pltpu.CompilerParams(needs_layout_passes=False),
        out_type=jax.ShapeDtypeStruct((N_NODE * 3,), jnp.float32),
        scratch_types=[
            pltpu.VMEM((N_NODE * 3,), jnp.float32),
            pltpu.VMEM((r_per_w * K,), jnp.int32),
            pltpu.VMEM((r_per_w * K,), jnp.float32),
            pltpu.VMEM((K * r_per_w,), jnp.int32),
            pltpu.VMEM((K * r_per_w,), jnp.float32),
            pltpu.VMEM((r_per_w * 3,), jnp.float32),
        ],
    )
    def sc_apply(pos_hbm, idx_hbm, s_hbm, out_hbm,
                 pos_v, idxr_v, sr_v, idxT_v, sT_v, out_v):
        wid = lax.axis_index("s") * 2 + lax.axis_index("c")
        base = wid * r_per_w
        pltpu.sync_copy(pos_hbm, pos_v)
        pltpu.sync_copy(idx_hbm.at[pl.ds(base * K, r_per_w * K)], idxr_v)
        pltpu.sync_copy(s_hbm.at[pl.ds(base * K, r_per_w * K)], sr_v)
        lanes = lax.broadcasted_iota(jnp.int32, (16,), 0)
        for r in range(r_per_w):
            tix = lanes * r_per_w + r
            plsc.store_scatter(idxT_v, [tix], idxr_v[pl.ds(r * K, K)])
            plsc.store_scatter(sT_v, [tix], sr_v[pl.ds(r * K, K)])
        inv = jnp.float32(1.0 / K)
        for g in range(r_per_w // 16):
            r0 = g * 16
            rows3 = (lanes + (base + r0)) * 3
            pxr = plsc.load_gather(pos_v, [rows3])
            pyr = plsc.load_gather(pos_v, [rows3 + 1])
            pzr = plsc.load_gather(pos_v, [rows3 + 2])
            accx = jnp.zeros((16,), jnp.float32)
            accy = jnp.zeros((16,), jnp.float32)
            accz = jnp.zeros((16,), jnp.float32)
            for j in range(K):
                off = j * r_per_w + r0
                iv3 = idxT_v[pl.ds(off, 16)] * 3
                sv = sT_v[pl.ds(off, 16)]
                gx = plsc.load_gather(pos_v, [iv3])
                gy = plsc.load_gather(pos_v, [iv3 + 1])
                gz = plsc.load_gather(pos_v, [iv3 + 2])
                accx = accx + (pxr - gx) * sv
                accy = accy + (pyr - gy) * sv
                accz = accz + (pzr - gz) * sv
            lrows3 = (lanes + r0) * 3
            plsc.store_scatter(out_v, [lrows3], pxr + accx * inv)
            plsc.store_scatter(out_v, [lrows3 + 1], pyr + accy * inv)
            plsc.store_scatter(out_v, [lrows3 + 2], pzr + accz * inv)
        pltpu.sync_copy(out_v, out_hbm.at[pl.ds(base * 3, r_per_w * 3)])

    return sc_apply


def kernel(pos, t, W0, b0, W1, b1, W2, b2):
    posT = pos.T
    w0a = W0[:, 0].reshape(1, HIDDEN)
    c0 = (t * W0[:, 1] + b0).reshape(1, HIDDEN)
    w1t = W1.T.astype(jnp.bfloat16)
    b1r = b1.reshape(1, HIDDEN)
    w2r = W2.reshape(1, HIDDEN)
    b2r = b2.reshape(1, 1)
    grid = N_NODE // B
    idx, s = pl.pallas_call(
        _topk_mlp_body,
        grid=(grid,),
        compiler_params=pltpu.CompilerParams(
            vmem_limit_bytes=100 * 1024 * 1024),
        in_specs=[
            pl.BlockSpec((B, 3), lambda i: (i, 0)),
            pl.BlockSpec((3, N_NODE), lambda i: (0, 0)),
            pl.BlockSpec((1, HIDDEN), lambda i: (0, 0)),
            pl.BlockSpec((1, HIDDEN), lambda i: (0, 0)),
            pl.BlockSpec((HIDDEN, HIDDEN), lambda i: (0, 0)),
            pl.BlockSpec((1, HIDDEN), lambda i: (0, 0)),
            pl.BlockSpec((1, HIDDEN), lambda i: (0, 0)),
            pl.BlockSpec((1, 1), lambda i: (0, 0)),
        ],
        out_specs=[
            pl.BlockSpec((B, K), lambda i: (i, 0)),
            pl.BlockSpec((B, K), lambda i: (i, 0)),
        ],
        out_shape=[
            jax.ShapeDtypeStruct((N_NODE, K), jnp.int32),
            jax.ShapeDtypeStruct((N_NODE, K), jnp.float32),
        ],
    )(pos, posT, w0a, c0, w1t, b1r, w2r, b2r)

    out = _sc_apply_build()(pos.reshape(-1), idx.reshape(-1), s.reshape(-1))
    return out.reshape(N_NODE, 3)


# speedup vs baseline: 1.0878x; 1.0878x over previous
import functools

import jax
import jax.numpy as jnp
from jax import lax
from jax.experimental import pallas as pl
from jax.experimental.pallas import tpu as pltpu
from jax.experimental.pallas import tpu_sc as plsc

N_NODE = 4096
K = 16
HIDDEN = 128
B = 512
BIG = 0x7F000000


def _topk_mlp_body(pos_ref, posT_ref, w0a_ref, c0_ref, w1t_ref, b1_ref,
                   w2_ref, b2_ref, idx_ref, s_ref, *, row_off=0):
    i = pl.program_id(0)
    pb = pos_ref[...]
    pt = posT_ref[...]
    g = jnp.dot(pb, pt, preferred_element_type=jnp.float32)
    ni = jnp.sum(pb * pb, axis=1, keepdims=True)
    nj = jnp.sum(pt * pt, axis=0, keepdims=True)
    d = jnp.maximum((ni + nj) - (g + g), 0.0)
    db = lax.bitcast_convert_type(d, jnp.int32)
    col = lax.broadcasted_iota(jnp.int32, (B, N_NODE), 1)
    row = lax.broadcasted_iota(jnp.int32, (B, N_NODE), 0) + (i * B + row_off)
    dp = (db & jnp.int32(-4096)) | col
    dp = jnp.where(col == row, jnp.int32(BIG), dp)
    dpf = lax.bitcast_convert_type(dp, jnp.float32)
    bigf = lax.bitcast_convert_type(jnp.int32(BIG), jnp.float32)
    m = None
    ms = []
    ss = []
    for p in range(K):
        if p == 0:
            m = jnp.min(dpf, axis=1, keepdims=True)
        else:
            m = jnp.min(jnp.where(dpf > m, dpf, bigf), axis=1, keepdims=True)
        ms.append(m)
        mi = lax.bitcast_convert_type(m, jnp.int32)
        rad = lax.bitcast_convert_type(mi & jnp.int32(-4096), jnp.float32)
        h = rad * w0a_ref[...] + c0_ref[...]
        h = h * jax.nn.sigmoid(h)
        h = h * lax.rsqrt(jnp.mean(h * h, axis=1, keepdims=True) + 1e-6)
        h = jnp.dot(h.astype(jnp.bfloat16), w1t_ref[...],
                    preferred_element_type=jnp.float32) + b1_ref[...]
        h = h * jax.nn.sigmoid(h)
        h = h * lax.rsqrt(jnp.mean(h * h, axis=1, keepdims=True) + 1e-6)
        ss.append(jnp.sum(h * w2_ref[...], axis=1, keepdims=True) + b2_ref[...])
    v = lax.bitcast_convert_type(jnp.concatenate(ms, axis=1), jnp.int32)
    idx_ref[...] = v & jnp.int32(4095)
    s_ref[...] = jnp.concatenate(ss, axis=1)


def _sc_apply_build(n_rows=N_NODE, row_off=0):
    mesh = plsc.VectorSubcoreMesh(core_axis_name="c", subcore_axis_name="s")
    n_workers = 32
    r_per_w = n_rows // n_workers

    @functools.partial(
        pl.kernel, mesh=mesh,
        compiler_params=pltpu.CompilerParams(needs_layout_passes=False),
        out_type=jax.ShapeDtypeStruct((n_rows * 3,), jnp.float32),
        scratch_types=[
            pltpu.VMEM((N_NODE * 3,), jnp.float32),
            pltpu.VMEM((r_per_w * K,), jnp.int32),
            pltpu.VMEM((r_per_w * K,), jnp.float32),
            pltpu.VMEM((K * r_per_w,), jnp.int32),
            pltpu.VMEM((K * r_per_w,), jnp.float32),
            pltpu.VMEM((r_per_w * 3,), jnp.float32),
        ],
    )
    def sc_apply(pos_hbm, idx_hbm, s_hbm, out_hbm,
                 pos_v, idxr_v, sr_v, idxT_v, sT_v, out_v):
        wid = lax.axis_index("s") * 2 + lax.axis_index("c")
        base = wid * r_per_w
        pltpu.sync_copy(pos_hbm, pos_v)
        pltpu.sync_copy(idx_hbm.at[pl.ds(base * K, r_per_w * K)], idxr_v)
        pltpu.sync_copy(s_hbm.at[pl.ds(base * K, r_per_w * K)], sr_v)
        lanes = lax.broadcasted_iota(jnp.int32, (16,), 0)
        for r in range(r_per_w):
            tix = lanes * r_per_w + r
            plsc.store_scatter(idxT_v, [tix], idxr_v[pl.ds(r * K, K)])
            plsc.store_scatter(sT_v, [tix], sr_v[pl.ds(r * K, K)])
        inv = jnp.float32(1.0 / K)
        for g in range(r_per_w // 16):
            r0 = g * 16
            rows3 = (lanes + (row_off + base + r0)) * 3
            pxr = plsc.load_gather(pos_v, [rows3])
            pyr = plsc.load_gather(pos_v, [rows3 + 1])
            pzr = plsc.load_gather(pos_v, [rows3 + 2])
            accx = jnp.zeros((16,), jnp.float32)
            accy = jnp.zeros((16,), jnp.float32)
            accz = jnp.zeros((16,), jnp.float32)
            for j in range(K):
                off = j * r_per_w + r0
                iv3 = idxT_v[pl.ds(off, 16)] * 3
                sv = sT_v[pl.ds(off, 16)]
                gx = plsc.load_gather(pos_v, [iv3])
                gy = plsc.load_gather(pos_v, [iv3 + 1])
                gz = plsc.load_gather(pos_v, [iv3 + 2])
                accx = accx + (pxr - gx) * sv
                accy = accy + (pyr - gy) * sv
                accz = accz + (pzr - gz) * sv
            lrows3 = (lanes + r0) * 3
            plsc.store_scatter(out_v, [lrows3], pxr + accx * inv)
            plsc.store_scatter(out_v, [lrows3 + 1], pyr + accy * inv)
            plsc.store_scatter(out_v, [lrows3 + 2], pzr + accz * inv)
        pltpu.sync_copy(out_v, out_hbm.at[pl.ds(base * 3, r_per_w * 3)])

    return sc_apply


def kernel(pos, t, W0, b0, W1, b1, W2, b2):
    posT = pos.T
    w0a = W0[:, 0].reshape(1, HIDDEN)
    c0 = (t * W0[:, 1] + b0).reshape(1, HIDDEN)
    w1t = W1.T.astype(jnp.bfloat16)
    b1r = b1.reshape(1, HIDDEN)
    w2r = W2.reshape(1, HIDDEN)
    b2r = b2.reshape(1, 1)
    half = N_NODE // 2
    pos_flat = pos.reshape(-1)
    outs = []
    for h in range(2):
        ph = lax.slice_in_dim(pos, h * half, (h + 1) * half)
        idx, s = pl.pallas_call(
            functools.partial(_topk_mlp_body, row_off=h * half),
            grid=(half // B,),
            compiler_params=pltpu.CompilerParams(
                dimension_semantics=("parallel",),
                vmem_limit_bytes=100 * 1024 * 1024),
            in_specs=[
                pl.BlockSpec((B, 3), lambda i: (i, 0)),
                pl.BlockSpec((3, N_NODE), lambda i: (0, 0)),
                pl.BlockSpec((1, HIDDEN), lambda i: (0, 0)),
                pl.BlockSpec((1, HIDDEN), lambda i: (0, 0)),
                pl.BlockSpec((HIDDEN, HIDDEN), lambda i: (0, 0)),
                pl.BlockSpec((1, HIDDEN), lambda i: (0, 0)),
                pl.BlockSpec((1, HIDDEN), lambda i: (0, 0)),
                pl.BlockSpec((1, 1), lambda i: (0, 0)),
            ],
            out_specs=[
                pl.BlockSpec((B, K), lambda i: (i, 0)),
                pl.BlockSpec((B, K), lambda i: (i, 0)),
            ],
            out_shape=[
                jax.ShapeDtypeStruct((half, K), jnp.int32),
                jax.ShapeDtypeStruct((half, K), jnp.float32),
            ],
        )(ph, posT, w0a, c0, w1t, b1r, w2r, b2r)
        outs.append(_sc_apply_build(half, h * half)(
            pos_flat, idx.reshape(-1), s.reshape(-1)))
    return jnp.concatenate(outs).reshape(N_NODE, 3)


# speedup vs baseline: 1.1809x; 1.0856x over previous
import functools

import jax
import jax.numpy as jnp
from jax import lax
from jax.experimental import pallas as pl
from jax.experimental.pallas import tpu as pltpu
from jax.experimental.pallas import tpu_sc as plsc

N_NODE = 4096
K = 16
HIDDEN = 128
B = 512
BIG = 0x7F000000


def _topk_mlp_body(pos_ref, posT_ref, w0a_ref, c0_ref, w1t_ref, b1_ref,
                   w2_ref, b2_ref, idx_ref, s_ref):
    i = pl.program_id(0)
    pb = pos_ref[...]
    pt = posT_ref[...]
    g = jnp.dot(pb, pt, preferred_element_type=jnp.float32)
    ni = jnp.sum(pb * pb, axis=1, keepdims=True)
    nj = jnp.sum(pt * pt, axis=0, keepdims=True)
    d = jnp.maximum((ni + nj) - (g + g), 0.0)
    db = lax.bitcast_convert_type(d, jnp.int32)
    col = lax.broadcasted_iota(jnp.int32, (B, N_NODE), 1)
    row = lax.broadcasted_iota(jnp.int32, (B, N_NODE), 0) + i * B
    dp = (db & jnp.int32(-4096)) | col
    dp = jnp.where(col == row, jnp.int32(BIG), dp)
    dpf = lax.bitcast_convert_type(dp, jnp.float32)
    bigf = lax.bitcast_convert_type(jnp.int32(BIG), jnp.float32)
    m = None
    ms = []
    ss = []
    for p in range(K):
        if p == 0:
            m = jnp.min(dpf, axis=1, keepdims=True)
        else:
            m = jnp.min(jnp.where(dpf > m, dpf, bigf), axis=1, keepdims=True)
        ms.append(m)
        mi = lax.bitcast_convert_type(m, jnp.int32)
        rad = lax.bitcast_convert_type(mi & jnp.int32(-4096), jnp.float32)
        h = rad * w0a_ref[...] + c0_ref[...]
        h = h * jax.nn.sigmoid(h)
        h = h * lax.rsqrt(jnp.mean(h * h, axis=1, keepdims=True) + 1e-6)
        h = jnp.dot(h.astype(jnp.bfloat16), w1t_ref[...],
                    preferred_element_type=jnp.float32) + b1_ref[...]
        h = h * jax.nn.sigmoid(h)
        h = h * lax.rsqrt(jnp.mean(h * h, axis=1, keepdims=True) + 1e-6)
        ss.append(jnp.sum(h * w2_ref[...], axis=1, keepdims=True) + b2_ref[...])
    v = lax.bitcast_convert_type(jnp.concatenate(ms, axis=1), jnp.int32)
    idx_ref[...] = v & jnp.int32(4095)
    s_ref[...] = jnp.concatenate(ss, axis=1)


def _sc_apply_build():
    mesh = plsc.VectorSubcoreMesh(core_axis_name="c", subcore_axis_name="s")
    n_workers = 32
    r_per_w = N_NODE // n_workers

    @functools.partial(
        pl.kernel, mesh=mesh,
        compiler_params=pltpu.CompilerParams(needs_layout_passes=False),
        out_type=jax.ShapeDtypeStruct((N_NODE * 3,), jnp.float32),
        scratch_types=[
            pltpu.VMEM((N_NODE * 3,), jnp.float32),
            pltpu.VMEM((r_per_w * K,), jnp.int32),
            pltpu.VMEM((r_per_w * K,), jnp.float32),
            pltpu.VMEM((K * r_per_w,), jnp.int32),
            pltpu.VMEM((K * r_per_w,), jnp.float32),
            pltpu.VMEM((r_per_w * 3,), jnp.float32),
        ],
    )
    def sc_apply(pos_hbm, idx_hbm, s_hbm, out_hbm,
                 pos_v, idxr_v, sr_v, idxT_v, sT_v, out_v):
        wid = lax.axis_index("s") * 2 + lax.axis_index("c")
        base = wid * r_per_w
        pltpu.sync_copy(pos_hbm, pos_v)
        pltpu.sync_copy(idx_hbm.at[pl.ds(base * K, r_per_w * K)], idxr_v)
        pltpu.sync_copy(s_hbm.at[pl.ds(base * K, r_per_w * K)], sr_v)
        lanes = lax.broadcasted_iota(jnp.int32, (16,), 0)
        for r in range(r_per_w):
            tix = lanes * r_per_w + r
            plsc.store_scatter(idxT_v, [tix], idxr_v[pl.ds(r * K, K)])
            plsc.store_scatter(sT_v, [tix], sr_v[pl.ds(r * K, K)])
        inv = jnp.float32(1.0 / K)
        for g in range(r_per_w // 16):
            r0 = g * 16
            rows3 = (lanes + (base + r0)) * 3
            pxr = plsc.load_gather(pos_v, [rows3])
            pyr = plsc.load_gather(pos_v, [rows3 + 1])
            pzr = plsc.load_gather(pos_v, [rows3 + 2])
            accx = jnp.zeros((16,), jnp.float32)
            accy = jnp.zeros((16,), jnp.float32)
            accz = jnp.zeros((16,), jnp.float32)
            for j in range(K):
                off = j * r_per_w + r0
                iv3 = idxT_v[pl.ds(off, 16)] * 3
                sv = sT_v[pl.ds(off, 16)]
                gx = plsc.load_gather(pos_v, [iv3])
                gy = plsc.load_gather(pos_v, [iv3 + 1])
                gz = plsc.load_gather(pos_v, [iv3 + 2])
                accx = accx + (pxr - gx) * sv
                accy = accy + (pyr - gy) * sv
                accz = accz + (pzr - gz) * sv
            lrows3 = (lanes + r0) * 3
            plsc.store_scatter(out_v, [lrows3], pxr + accx * inv)
            plsc.store_scatter(out_v, [lrows3 + 1], pyr + accy * inv)
            plsc.store_scatter(out_v, [lrows3 + 2], pzr + accz * inv)
        pltpu.sync_copy(out_v, out_hbm.at[pl.ds(base * 3, r_per_w * 3)])

    return sc_apply


def kernel(pos, t, W0, b0, W1, b1, W2, b2):
    posT = pos.T
    w0a = W0[:, 0].reshape(1, HIDDEN)
    c0 = (t * W0[:, 1] + b0).reshape(1, HIDDEN)
    w1t = W1.T.astype(jnp.bfloat16)
    b1r = b1.reshape(1, HIDDEN)
    w2r = W2.reshape(1, HIDDEN)
    b2r = b2.reshape(1, 1)
    grid = N_NODE // B
    idx, s = pl.pallas_call(
        _topk_mlp_body,
        grid=(grid,),
        compiler_params=pltpu.CompilerParams(
            dimension_semantics=("parallel",),
            vmem_limit_bytes=100 * 1024 * 1024),
        in_specs=[
            pl.BlockSpec((B, 3), lambda i: (i, 0)),
            pl.BlockSpec((3, N_NODE), lambda i: (0, 0)),
            pl.BlockSpec((1, HIDDEN), lambda i: (0, 0)),
            pl.BlockSpec((1, HIDDEN), lambda i: (0, 0)),
            pl.BlockSpec((HIDDEN, HIDDEN), lambda i: (0, 0)),
            pl.BlockSpec((1, HIDDEN), lambda i: (0, 0)),
            pl.BlockSpec((1, HIDDEN), lambda i: (0, 0)),
            pl.BlockSpec((1, 1), lambda i: (0, 0)),
        ],
        out_specs=[
            pl.BlockSpec((B, K), lambda i: (i, 0)),
            pl.BlockSpec((B, K), lambda i: (i, 0)),
        ],
        out_shape=[
            jax.ShapeDtypeStruct((N_NODE, K), jnp.int32),
            jax.ShapeDtypeStruct((N_NODE, K), jnp.float32),
        ],
    )(pos, posT, w0a, c0, w1t, b1r, w2r, b2r)

    out = _sc_apply_build()(pos.reshape(-1), idx.reshape(-1), s.reshape(-1))
    return out.reshape(N_NODE, 3)
